# baseline (speedup 1.0000x reference)
import functools

import jax
import jax.numpy as jnp
from jax import lax
from jax.experimental import pallas as pl
from jax.experimental.pallas import tpu as pltpu
from jax.experimental.pallas import tpu_sc as plsc

N = 10000
D = 128
H = 64
G = 64
C = 10

NC = 2
NS = 16
NW = NC * NS

CHUNK = 128
GROUP = 8
ROWS_PER_TILE = 632
N_PAD = ROWS_PER_TILE * NS

_HIGH = jax.lax.Precision.HIGHEST


def _sc_mesh():
    return plsc.VectorSubcoreMesh(core_axis_name="c", subcore_axis_name="s")


def _make_sc_count(nchunks):
    @functools.partial(
        pl.kernel, mesh=_sc_mesh(),
        compiler_params=pltpu.CompilerParams(use_tc_tiling_on_sc=False),
        out_type=jax.ShapeDtypeStruct((NC, N_PAD, 8), jnp.float32),
        scratch_types=[
            pltpu.VMEM((nchunks, CHUNK), jnp.int32),
            pltpu.VMEM((CHUNK, 8), jnp.float32),
            pltpu.VMEM_SHARED((N_PAD, 8), jnp.float32),
            pltpu.SemaphoreType.DMA,
        ],
    )
    def sc_count(dst_hbm, zeros_hbm, ones_hbm, out_hbm, didx_v, ones_v,
                 acc_s, sem):
        c = lax.axis_index("c")
        s = lax.axis_index("s")
        row0 = s * ROWS_PER_TILE
        pltpu.sync_copy(zeros_hbm, acc_s.at[pl.ds(row0, ROWS_PER_TILE)])
        pltpu.sync_copy(ones_hbm, ones_v)
        rbase = (c * NS + s) * nchunks
        pltpu.sync_copy(dst_hbm.at[pl.ds(rbase, nchunks)], didx_v)
        plsc.subcore_barrier()

        def fire(i, carry):
            pltpu.async_copy(ones_v, acc_s.at[didx_v.at[i]], sem, add=True)
            return carry

        lax.fori_loop(0, nchunks, fire, 0)

        def drain(i, carry):
            pltpu.make_async_copy(ones_v, acc_s.at[didx_v.at[i]], sem).wait()
            return carry

        lax.fori_loop(0, nchunks, drain, 0)
        plsc.subcore_barrier()
        pltpu.sync_copy(acc_s.at[pl.ds(row0, ROWS_PER_TILE)],
                        out_hbm.at[c, pl.ds(row0, ROWS_PER_TILE)])

    return sc_count


def _make_sc_edge(nchunks):
    @functools.partial(
        pl.kernel, mesh=_sc_mesh(),
        compiler_params=pltpu.CompilerParams(use_tc_tiling_on_sc=False),
        out_type=jax.ShapeDtypeStruct((NC, N_PAD, H), jnp.float32),
        scratch_types=[
            pltpu.VMEM((nchunks, CHUNK), jnp.int32),
            pltpu.VMEM((nchunks, CHUNK), jnp.int32),
            [pltpu.VMEM((CHUNK, H), jnp.float32)] * GROUP,
            [pltpu.SemaphoreType.DMA] * GROUP,
            [pltpu.SemaphoreType.DMA] * GROUP,
            pltpu.VMEM_SHARED((N_PAD, H), jnp.float32),
        ],
    )
    def sc_edge(src_hbm, dst_hbm, q_hbm, zeros_hbm, out_hbm,
                sidx_v, didx_v, bufs, gsems, ssems, acc_s):
        c = lax.axis_index("c")
        s = lax.axis_index("s")
        row0 = s * ROWS_PER_TILE
        pltpu.sync_copy(zeros_hbm, acc_s.at[pl.ds(row0, ROWS_PER_TILE)])
        rbase = (c * NS + s) * nchunks
        pltpu.sync_copy(src_hbm.at[pl.ds(rbase, nchunks)], sidx_v)
        pltpu.sync_copy(dst_hbm.at[pl.ds(rbase, nchunks)], didx_v)
        plsc.subcore_barrier()

        def body(g, carry):
            i0 = g * GROUP
            gds = []
            for b in range(GROUP):
                gds.append(pltpu.async_copy(
                    q_hbm.at[sidx_v.at[i0 + b]], bufs[b], gsems[b]))
            sds = []
            for b in range(GROUP):
                gds[b].wait()
                sds.append(pltpu.async_copy(
                    bufs[b], acc_s.at[didx_v.at[i0 + b]], ssems[b], add=True))
            for b in range(GROUP):
                sds[b].wait()
            return carry

        lax.fori_loop(0, nchunks // GROUP, body, 0)
        plsc.subcore_barrier()
        pltpu.sync_copy(acc_s.at[pl.ds(row0, ROWS_PER_TILE)],
                        out_hbm.at[c, pl.ds(row0, ROWS_PER_TILE)])

    return sc_edge


BLK = 1000
GRID = N // BLK


def _tc_a_body(cnt_ref, x_ref, w1_ref, dis_ref, q1_ref):
    cnt = cnt_ref[...]
    deg = 1.0 + cnt[0, :, 0] + cnt[1, :, 0]
    dis = jax.lax.rsqrt(deg)
    p = lax.dot_general(x_ref[...], w1_ref[...], (((1,), (0,)), ((), ())),
                        precision=_HIGH, preferred_element_type=jnp.float32)
    q1_ref[...] = dis[:, None] * p
    dis_ref[...] = jnp.broadcast_to(dis[:, None], (BLK, 8))


def _tc_b_body(agg_ref, q1_ref, dis_ref, b1_ref, w2_ref, q2_ref):
    agg = agg_ref[0] + agg_ref[1] + q1_ref[...]
    dis = dis_ref[...][:, :1]
    out1 = jnp.maximum(dis * agg + b1_ref[...], 0.0)
    p2 = lax.dot_general(out1, w2_ref[...], (((1,), (0,)), ((), ())),
                         precision=_HIGH, preferred_element_type=jnp.float32)
    q2_ref[...] = dis * p2


def _tc_c_body(agg_ref, q2_ref, dis_ref, b2_ref, batch_ref, wf_ref, bf_ref,
               out_ref, acc_ref):
    i = pl.program_id(0)
    agg = agg_ref[0] + agg_ref[1] + q2_ref[...]
    dis = dis_ref[...][:, :1]
    out2 = jnp.maximum(dis * agg + b2_ref[...], 0.0)
    b = batch_ref[0, 0, :]
    onehot = (b[:, None] == lax.broadcasted_iota(jnp.int32, (BLK, G), 1)
              ).astype(jnp.float32)
    contrib = lax.dot_general(onehot, out2, (((0,), (0,)), ((), ())),
                              precision=_HIGH, preferred_element_type=jnp.float32)

    @pl.when(i == 0)
    def _():
        acc_ref[...] = contrib

    @pl.when(i > 0)
    def _():
        acc_ref[...] = acc_ref[...] + contrib

    @pl.when(i == pl.num_programs(0) - 1)
    def _():
        logits = lax.dot_general(acc_ref[...], wf_ref[...], (((1,), (0,)), ((), ())),
                                 precision=_HIGH, preferred_element_type=jnp.float32)
        logits = logits + bf_ref[...]
        col = lax.broadcasted_iota(jnp.int32, (G, 128), 1)
        logits = jnp.where(col < C, logits, -1e30)
        m = jnp.max(logits, axis=1, keepdims=True)
        lse = m + jnp.log(jnp.sum(jnp.exp(logits - m), axis=1, keepdims=True))
        out_ref[...] = logits - lse


def kernel(x, edge_index, batch, W1, b1, W2, b2, Wf, bf):
    E = edge_index.shape[1]
    step = CHUNK * GROUP
    ept = ((E + NW - 1) // NW + step - 1) // step * step
    e_pad = ept * NW
    nchunks = ept // CHUNK
    src = jnp.concatenate(
        [edge_index[0], jnp.zeros((e_pad - E,), dtype=jnp.int32)]
    ).reshape(NW * nchunks, CHUNK)
    dst = jnp.concatenate(
        [edge_index[1], jnp.full((e_pad - E,), N_PAD - 1, dtype=jnp.int32)]
    ).reshape(NW * nchunks, CHUNK)

    zeros8 = jnp.zeros((ROWS_PER_TILE, 8), jnp.float32)
    ones8 = jnp.ones((CHUNK, 8), jnp.float32)
    zerosH = jnp.zeros((ROWS_PER_TILE, H), jnp.float32)

    sc_count = _make_sc_count(nchunks)
    sc_edge = _make_sc_edge(nchunks)

    cnt = sc_count(dst, zeros8, ones8)

    dis8, q1 = pl.pallas_call(
        _tc_a_body,
        grid=(GRID,),
        in_specs=[
            pl.BlockSpec((NC, BLK, 8), lambda i: (0, i, 0)),
            pl.BlockSpec((BLK, D), lambda i: (i, 0)),
            pl.BlockSpec((D, H), lambda i: (0, 0)),
        ],
        out_specs=[
            pl.BlockSpec((BLK, 8), lambda i: (i, 0)),
            pl.BlockSpec((BLK, H), lambda i: (i, 0)),
        ],
        out_shape=[
            jax.ShapeDtypeStruct((N, 8), jnp.float32),
            jax.ShapeDtypeStruct((N, H), jnp.float32),
        ],
    )(cnt, x, W1)

    agg1 = sc_edge(src, dst, q1, zerosH)

    q2 = pl.pallas_call(
        _tc_b_body,
        grid=(GRID,),
        in_specs=[
            pl.BlockSpec((NC, BLK, H), lambda i: (0, i, 0)),
            pl.BlockSpec((BLK, H), lambda i: (i, 0)),
            pl.BlockSpec((BLK, 8), lambda i: (i, 0)),
            pl.BlockSpec((1, H), lambda i: (0, 0)),
            pl.BlockSpec((H, H), lambda i: (0, 0)),
        ],
        out_specs=pl.BlockSpec((BLK, H), lambda i: (i, 0)),
        out_shape=jax.ShapeDtypeStruct((N, H), jnp.float32),
    )(agg1, q1, dis8, b1[None, :], W2)

    agg2 = sc_edge(src, dst, q2, zerosH)

    batch_r = batch.reshape(GRID, 1, BLK)
    wf_pad = jnp.zeros((H, 128), jnp.float32).at[:, :C].set(Wf)
    bf_pad = jnp.zeros((1, 128), jnp.float32).at[0, :C].set(bf)

    out_pad = pl.pallas_call(
        _tc_c_body,
        grid=(GRID,),
        in_specs=[
            pl.BlockSpec((NC, BLK, H), lambda i: (0, i, 0)),
            pl.BlockSpec((BLK, H), lambda i: (i, 0)),
            pl.BlockSpec((BLK, 8), lambda i: (i, 0)),
            pl.BlockSpec((1, H), lambda i: (0, 0)),
            pl.BlockSpec((1, 1, BLK), lambda i: (i, 0, 0)),
            pl.BlockSpec((H, 128), lambda i: (0, 0)),
            pl.BlockSpec((1, 128), lambda i: (0, 0)),
        ],
        out_specs=pl.BlockSpec((G, 128), lambda i: (0, 0)),
        out_shape=jax.ShapeDtypeStruct((G, 128), jnp.float32),
        scratch_shapes=[pltpu.VMEM((G, G), jnp.float32)],
    )(agg2, q2, dis8, b2[None, :], batch_r, wf_pad, bf_pad)

    return out_pad[:, :C]


# speedup vs baseline: 2.5170x; 2.5170x over previous
import functools

import jax
import jax.numpy as jnp
from jax import lax
from jax.experimental import pallas as pl
from jax.experimental.pallas import tpu as pltpu
from jax.experimental.pallas import tpu_sc as plsc

N = 10000
D = 128
H = 64
G = 64
C = 10

NC = 2
NS = 16
NW = NC * NS

CHUNK = 128
GROUP = 8
ROWS_PER_TILE = 632
N_PAD = ROWS_PER_TILE * NS

_HIGH = jax.lax.Precision.HIGHEST


def _sc_mesh():
    return plsc.VectorSubcoreMesh(core_axis_name="c", subcore_axis_name="s")


def _make_sc_count(nchunks):
    @functools.partial(
        pl.kernel, mesh=_sc_mesh(),
        compiler_params=pltpu.CompilerParams(use_tc_tiling_on_sc=False),
        out_type=jax.ShapeDtypeStruct((NC, N_PAD, 8), jnp.float32),
        scratch_types=[
            pltpu.VMEM((nchunks, CHUNK), jnp.int32),
            pltpu.VMEM((CHUNK, 8), jnp.float32),
            pltpu.VMEM_SHARED((N_PAD, 8), jnp.float32),
            pltpu.SemaphoreType.DMA,
        ],
    )
    def sc_count(dst_hbm, zeros_hbm, ones_hbm, out_hbm, didx_v, ones_v,
                 acc_s, sem):
        c = lax.axis_index("c")
        s = lax.axis_index("s")
        row0 = s * ROWS_PER_TILE
        pltpu.sync_copy(zeros_hbm, acc_s.at[pl.ds(row0, ROWS_PER_TILE)])
        pltpu.sync_copy(ones_hbm, ones_v)
        rbase = (c * NS + s) * nchunks
        pltpu.sync_copy(dst_hbm.at[pl.ds(rbase, nchunks)], didx_v)
        plsc.subcore_barrier()

        def fire(i, carry):
            pltpu.async_copy(ones_v, acc_s.at[didx_v.at[i]], sem, add=True)
            return carry

        lax.fori_loop(0, nchunks, fire, 0)

        def drain(i, carry):
            pltpu.make_async_copy(ones_v, acc_s.at[didx_v.at[i]], sem).wait()
            return carry

        lax.fori_loop(0, nchunks, drain, 0)
        plsc.subcore_barrier()
        pltpu.sync_copy(acc_s.at[pl.ds(row0, ROWS_PER_TILE)],
                        out_hbm.at[c, pl.ds(row0, ROWS_PER_TILE)])

    return sc_count


def _make_sc_edge(nchunks):
    @functools.partial(
        pl.kernel, mesh=_sc_mesh(),
        compiler_params=pltpu.CompilerParams(use_tc_tiling_on_sc=False),
        out_type=jax.ShapeDtypeStruct((NC, N_PAD, H), jnp.float32),
        scratch_types=[
            pltpu.VMEM((nchunks, CHUNK), jnp.int32),
            pltpu.VMEM((nchunks, CHUNK), jnp.int32),
            [pltpu.VMEM((CHUNK, H), jnp.float32)] * GROUP,
            [pltpu.SemaphoreType.DMA] * GROUP,
            [pltpu.SemaphoreType.DMA] * GROUP,
            pltpu.VMEM_SHARED((N_PAD, H), jnp.float32),
        ],
    )
    def sc_edge(src_hbm, dst_hbm, q_hbm, zeros_hbm, out_hbm,
                sidx_v, didx_v, bufs, gsems, ssems, acc_s):
        c = lax.axis_index("c")
        s = lax.axis_index("s")
        row0 = s * ROWS_PER_TILE
        pltpu.sync_copy(zeros_hbm, acc_s.at[pl.ds(row0, ROWS_PER_TILE)])
        rbase = (c * NS + s) * nchunks
        pltpu.sync_copy(src_hbm.at[pl.ds(rbase, nchunks)], sidx_v)
        pltpu.sync_copy(dst_hbm.at[pl.ds(rbase, nchunks)], didx_v)
        plsc.subcore_barrier()

        def body(g, carry):
            i0 = g * GROUP
            gds = []
            for b in range(GROUP):
                gds.append(pltpu.async_copy(
                    q_hbm.at[sidx_v.at[i0 + b]], bufs[b], gsems[b]))
            sds = []
            for b in range(GROUP):
                gds[b].wait()
                sds.append(pltpu.async_copy(
                    bufs[b], acc_s.at[didx_v.at[i0 + b]], ssems[b], add=True))
            for b in range(GROUP):
                sds[b].wait()
            return carry

        lax.fori_loop(0, nchunks // GROUP, body, 0)
        plsc.subcore_barrier()
        pltpu.sync_copy(acc_s.at[pl.ds(row0, ROWS_PER_TILE)],
                        out_hbm.at[c, pl.ds(row0, ROWS_PER_TILE)])

    return sc_edge


BLK = 1000
GRID = N // BLK


def _tc_a_body(cnt_ref, x_ref, w1_ref, dis_ref, q1_ref):
    cnt = cnt_ref[...]
    deg = 1.0 + cnt[0, :, 0] + cnt[1, :, 0]
    dis = jax.lax.rsqrt(deg)
    p = lax.dot_general(x_ref[...], w1_ref[...], (((1,), (0,)), ((), ())),
                        precision=_HIGH, preferred_element_type=jnp.float32)
    q1_ref[...] = dis[:, None] * p
    dis_ref[...] = jnp.broadcast_to(dis[:, None], (BLK, 8))


def _tc_b_body(agg_ref, q1_ref, dis_ref, b1_ref, w2_ref, q2_ref):
    agg = agg_ref[0] + agg_ref[1] + q1_ref[...]
    dis = dis_ref[...][:, :1]
    out1 = jnp.maximum(dis * agg + b1_ref[...], 0.0)
    p2 = lax.dot_general(out1, w2_ref[...], (((1,), (0,)), ((), ())),
                         precision=_HIGH, preferred_element_type=jnp.float32)
    q2_ref[...] = dis * p2


def _tc_c_body(agg_ref, q2_ref, dis_ref, b2_ref, batch_ref, wf_ref, bf_ref,
               out_ref, acc_ref):
    i = pl.program_id(0)
    agg = agg_ref[0] + agg_ref[1] + q2_ref[...]
    dis = dis_ref[...][:, :1]
    out2 = jnp.maximum(dis * agg + b2_ref[...], 0.0)
    b = batch_ref[0, 0, :]
    onehot = (b[:, None] == lax.broadcasted_iota(jnp.int32, (BLK, G), 1)
              ).astype(jnp.float32)
    contrib = lax.dot_general(onehot, out2, (((0,), (0,)), ((), ())),
                              precision=_HIGH, preferred_element_type=jnp.float32)

    @pl.when(i == 0)
    def _():
        acc_ref[...] = contrib

    @pl.when(i > 0)
    def _():
        acc_ref[...] = acc_ref[...] + contrib

    @pl.when(i == pl.num_programs(0) - 1)
    def _():
        logits = lax.dot_general(acc_ref[...], wf_ref[...], (((1,), (0,)), ((), ())),
                                 precision=_HIGH, preferred_element_type=jnp.float32)
        logits = logits + bf_ref[...]
        col = lax.broadcasted_iota(jnp.int32, (G, 128), 1)
        logits = jnp.where(col < C, logits, -1e30)
        m = jnp.max(logits, axis=1, keepdims=True)
        lse = m + jnp.log(jnp.sum(jnp.exp(logits - m), axis=1, keepdims=True))
        out_ref[...] = logits - lse


def kernel(x, edge_index, batch, W1, b1, W2, b2, Wf, bf):
    E = edge_index.shape[1]
    step = CHUNK * GROUP
    ept = ((E + NW - 1) // NW + step - 1) // step * step
    e_pad = ept * NW
    nchunks = ept // CHUNK
    npad = e_pad - E
    pad_dst = N + jnp.arange(npad, dtype=jnp.int32) % jnp.int32(N_PAD - N)
    pad_src = jnp.arange(npad, dtype=jnp.int32) % jnp.int32(N)
    src = jnp.concatenate([edge_index[0], pad_src]).reshape(
        NW * nchunks, CHUNK)
    dst = jnp.concatenate([edge_index[1], pad_dst]).reshape(
        NW * nchunks, CHUNK)

    zeros8 = jnp.zeros((ROWS_PER_TILE, 8), jnp.float32)
    ones8 = jnp.ones((CHUNK, 8), jnp.float32)
    zerosH = jnp.zeros((ROWS_PER_TILE, H), jnp.float32)

    sc_count = _make_sc_count(nchunks)
    sc_edge = _make_sc_edge(nchunks)

    cnt = sc_count(dst, zeros8, ones8)

    dis8, q1 = pl.pallas_call(
        _tc_a_body,
        grid=(GRID,),
        in_specs=[
            pl.BlockSpec((NC, BLK, 8), lambda i: (0, i, 0)),
            pl.BlockSpec((BLK, D), lambda i: (i, 0)),
            pl.BlockSpec((D, H), lambda i: (0, 0)),
        ],
        out_specs=[
            pl.BlockSpec((BLK, 8), lambda i: (i, 0)),
            pl.BlockSpec((BLK, H), lambda i: (i, 0)),
        ],
        out_shape=[
            jax.ShapeDtypeStruct((N, 8), jnp.float32),
            jax.ShapeDtypeStruct((N, H), jnp.float32),
        ],
    )(cnt, x, W1)

    agg1 = sc_edge(src, dst, q1, zerosH)

    q2 = pl.pallas_call(
        _tc_b_body,
        grid=(GRID,),
        in_specs=[
            pl.BlockSpec((NC, BLK, H), lambda i: (0, i, 0)),
            pl.BlockSpec((BLK, H), lambda i: (i, 0)),
            pl.BlockSpec((BLK, 8), lambda i: (i, 0)),
            pl.BlockSpec((1, H), lambda i: (0, 0)),
            pl.BlockSpec((H, H), lambda i: (0, 0)),
        ],
        out_specs=pl.BlockSpec((BLK, H), lambda i: (i, 0)),
        out_shape=jax.ShapeDtypeStruct((N, H), jnp.float32),
    )(agg1, q1, dis8, b1[None, :], W2)

    agg2 = sc_edge(src, dst, q2, zerosH)

    batch_r = batch.reshape(GRID, 1, BLK)
    wf_pad = jnp.zeros((H, 128), jnp.float32).at[:, :C].set(Wf)
    bf_pad = jnp.zeros((1, 128), jnp.float32).at[0, :C].set(bf)

    out_pad = pl.pallas_call(
        _tc_c_body,
        grid=(GRID,),
        in_specs=[
            pl.BlockSpec((NC, BLK, H), lambda i: (0, i, 0)),
            pl.BlockSpec((BLK, H), lambda i: (i, 0)),
            pl.BlockSpec((BLK, 8), lambda i: (i, 0)),
            pl.BlockSpec((1, H), lambda i: (0, 0)),
            pl.BlockSpec((1, 1, BLK), lambda i: (i, 0, 0)),
            pl.BlockSpec((H, 128), lambda i: (0, 0)),
            pl.BlockSpec((1, 128), lambda i: (0, 0)),
        ],
        out_specs=pl.BlockSpec((G, 128), lambda i: (0, 0)),
        out_shape=jax.ShapeDtypeStruct((G, 128), jnp.float32),
        scratch_shapes=[pltpu.VMEM((G, G), jnp.float32)],
    )(agg2, q2, dis8, b2[None, :], batch_r, wf_pad, bf_pad)

    return out_pad[:, :C]
